# Initial kernel scaffold; baseline (speedup 1.0000x reference)
import jax
import jax.numpy as jnp
from jax.experimental import pallas as pl


def kernel(x, routing_logits, batch_size, seq_len, W_up, b_up, W_down, b_down):
    raise NotImplementedError("write your pallas kernel here")


# speedup vs baseline: 1.0379x; 1.0379x over previous
import functools

import jax
import jax.numpy as jnp
from jax import lax
from jax.experimental import pallas as pl
from jax.experimental.pallas import tpu as pltpu
from jax.experimental.pallas import tpu_sc as plsc

_NUM_EXPERTS = 8
_GROUP = 2


def _make_sc_gather(n_rows, d, n_idx):
    info = plsc.get_sparse_core_info()
    nc, ns = info.num_cores, info.num_subcores
    nw = nc * ns
    assert n_idx % nw == 0
    per_w = n_idx // nw
    chunk = 64 if per_w % 64 == 0 else per_w
    n_chunks = per_w // chunk
    mesh = plsc.VectorSubcoreMesh(core_axis_name="c", subcore_axis_name="s")

    @functools.partial(
        pl.kernel,
        mesh=mesh,
        out_type=jax.ShapeDtypeStruct((n_idx, d), jnp.float32),
        scratch_types=[
            pltpu.VMEM((per_w,), jnp.int32),
            pltpu.VMEM((chunk, d), jnp.float32),
            pltpu.SemaphoreType.DMA,
        ],
    )
    def k(x_hbm, idx_hbm, out_hbm, idx_v, rows_v, sem):
        wid = lax.axis_index("s") * nc + lax.axis_index("c")
        base = wid * per_w
        pltpu.sync_copy(idx_hbm.at[pl.ds(base, per_w)], idx_v)
        for c in range(n_chunks):
            pltpu.async_copy(x_hbm.at[idx_v.at[pl.ds(c * chunk, chunk)]],
                             rows_v, sem).wait()
            pltpu.sync_copy(rows_v, out_hbm.at[pl.ds(base + c * chunk, chunk)])

    return k


def _ffn_body(tok_ref, g_ref, wup_ref, bup_ref, wdn_ref, bdn_ref, out_ref):
    t = tok_ref[0]
    up = jnp.dot(t.astype(jnp.bfloat16), wup_ref[0].astype(jnp.bfloat16),
                 preferred_element_type=jnp.float32) + bup_ref[0]
    a = up * jax.nn.sigmoid(up)
    dn = jnp.dot(a.astype(jnp.bfloat16), wdn_ref[0].astype(jnp.bfloat16),
                 preferred_element_type=jnp.float32) + bdn_ref[0]
    out_ref[0] = g_ref[0] * dn


def _tc_ffn(tokens, gates, w_up, b_up, w_dn, b_dn):
    e, kk, h = tokens.shape
    ff = w_up.shape[-1]
    return pl.pallas_call(
        _ffn_body,
        grid=(e,),
        in_specs=[
            pl.BlockSpec((1, kk, h), lambda i: (i, 0, 0)),
            pl.BlockSpec((1, kk, 1), lambda i: (i, 0, 0)),
            pl.BlockSpec((1, h, ff), lambda i: (i, 0, 0)),
            pl.BlockSpec((1, 1, ff), lambda i: (i, 0, 0)),
            pl.BlockSpec((1, ff, h), lambda i: (i // _GROUP, 0, 0)),
            pl.BlockSpec((1, 1, h), lambda i: (i // _GROUP, 0, 0)),
        ],
        out_specs=pl.BlockSpec((1, kk, h), lambda i: (i, 0, 0)),
        out_shape=jax.ShapeDtypeStruct((e, kk, h), jnp.float32),
    )(tokens, gates, w_up, b_up[:, None, :], w_dn, b_dn[:, None, :])


def kernel(x, routing_logits, batch_size, seq_len, W_up, b_up, W_down, b_down):
    bs, hidden = x.shape
    k = min(bs // _NUM_EXPERTS, bs)
    s = jax.nn.softmax(routing_logits, axis=-1)
    g_t, idx_t = jax.lax.top_k(s.T, k)
    flat_idx = idx_t.reshape(-1).astype(jnp.int32)

    tokens = _make_sc_gather(bs, hidden, _NUM_EXPERTS * k)(x, flat_idx)
    tokens = tokens.reshape(_NUM_EXPERTS, k, hidden)

    out = _tc_ffn(tokens, g_t.reshape(_NUM_EXPERTS, k, 1),
                  W_up, b_up, W_down, b_down)

    y = jnp.zeros((bs, hidden), dtype=x.dtype).at[flat_idx].add(
        out.reshape(-1, hidden))
    return y
